# baseline (speedup 1.0000x reference)
import jax
import jax.numpy as jnp
from jax.experimental import pallas as pl
from jax.experimental.pallas import tpu as pltpu

_N = 32768
_D = 256
_DOUT = 128
_NSEG = 16
_NLAYERS = 3
_B = 4096
_NB = _N // _B
_EPS = 1e-5
_F32 = jnp.float32


def _mlp_rows(x, w1, b1, w2, b2):
    h = jnp.maximum(_bdot(x, w1) + b1, 0.0)
    return _bdot(h, w2) + b2


def _mlp_rows_hi(x, w1, b1, w2, b2):
    h = jnp.maximum(_bdot(x, w1) + b1, 0.0)
    return _bdot(h, w2) + b2


def _bdot(a, b):
    return jnp.dot(a.astype(jnp.bfloat16), b.astype(jnp.bfloat16),
                   preferred_element_type=_F32)


def _onehot_t(seg_ref):
    sv = seg_ref[0]
    ids = jax.lax.broadcasted_iota(jnp.int32, (_NSEG, _B), 0)
    ot = jnp.where(ids == sv, 1.0, 0.0).astype(_F32)
    return ot.astype(jnp.bfloat16)


def _split(v):
    hi = v.astype(jnp.bfloat16)
    lo = (v - hi.astype(_F32)).astype(jnp.bfloat16)
    return hi, lo


def _otdot(ot, v):
    hi, lo = _split(v)
    return (jnp.dot(ot, lo, preferred_element_type=_F32) +
            jnp.dot(ot, hi, preferred_element_type=_F32))


def _otdot_hi(ot, v):
    return jnp.dot(ot, v.astype(jnp.bfloat16), preferred_element_type=_F32)


def _accum(i, ref, val):
    @pl.when(i == 0)
    def _():
        ref[...] = val

    @pl.when(i > 0)
    def _():
        ref[...] += val


def _stats_step0(i, ssh_in, ssp_in, sq_in, cnt_in, rw1, rb1, rw2, rb2,
                 bng, bnb, rhi_s, rlo_s, scale_s, shift_s):
    @pl.when(i == 0)
    def _():
        s = jnp.sum(ssh_in[...], axis=0)
        r = _mlp_rows_hi(s, rw1[...], rb1[...], rw2[...], rb2[...])
        c = jnp.sum(cnt_in[...], axis=0)[:, :1]
        g = jnp.sum(ssp_in[...], axis=0)
        s1 = jnp.sum(g + c * r, axis=0, keepdims=True)
        s2 = (jnp.sum(sq_in[...], axis=0) +
              jnp.sum((2.0 * g + c * r) * r, axis=0, keepdims=True))
        mean = s1 / _N
        var = s2 / _N - mean * mean
        sc = bng[...] / jnp.sqrt(var + _EPS)
        rhi, rlo = _split(r)
        rhi_s[...] = rhi
        rlo_s[...] = rlo
        scale_s[...] = sc
        shift_s[...] = bnb[...] - mean * sc


def _apply_bn(h_ref, xphi_ref, ot, rhi_s, rlo_s, scale_s, shift_s):
    dn = (((0,), (0,)), ((), ()))
    rr = (jax.lax.dot_general(ot, rlo_s[...], dn, preferred_element_type=_F32) +
          jax.lax.dot_general(ot, rhi_s[...], dn, preferred_element_type=_F32))
    t = xphi_ref[...].astype(_F32) + rr
    return h_ref[...] + t * scale_s[...] + shift_s[...]


def _first_kernel(x_ref, seg_ref, w1, b1, w2, b2,
                  xphi_out, ssh_out, ssp_out, sq_out, cnt_out):
    ot = _onehot_t(seg_ref)
    x = x_ref[...]
    xp = _mlp_rows(x, w1[...], b1[...], w2[...], b2[...])
    xphi_out[...] = xp.astype(jnp.bfloat16)
    ssh_out[0] = _otdot(ot, x)
    ssp_out[0] = _otdot_hi(ot, xp)
    sq_out[0] = jnp.sum(xp * xp, axis=0, keepdims=True)
    cnt_out[0] = jnp.broadcast_to(
        jnp.sum(ot.astype(_F32), axis=1, keepdims=True), (_NSEG, 128))


def _mid_kernel(h_ref, xphi_ref, seg_ref,
                ssh_in, ssp_in, sq_in, cnt_in,
                rw1, rb1, rw2, rb2, bng, bnb,
                pw1, pb1, pw2, pb2,
                h_out, xphi_out, ssh_out, ssp_out, sq_out,
                rhi_s, rlo_s, scale_s, shift_s):
    i = pl.program_id(0)
    _stats_step0(i, ssh_in, ssp_in, sq_in, cnt_in, rw1, rb1, rw2, rb2,
                 bng, bnb, rhi_s, rlo_s, scale_s, shift_s)
    ot = _onehot_t(seg_ref)
    hn = _apply_bn(h_ref, xphi_ref, ot, rhi_s, rlo_s, scale_s, shift_s)
    h_out[...] = hn
    xp = _mlp_rows(hn, pw1[...], pb1[...], pw2[...], pb2[...])
    xphi_out[...] = xp.astype(jnp.bfloat16)
    ssh_out[0] = _otdot(ot, hn)
    ssp_out[0] = _otdot_hi(ot, xp)
    sq_out[0] = jnp.sum(xp * xp, axis=0, keepdims=True)


def _final_kernel(h_ref, xphi_ref, seg_ref,
                  rhi, rlo, scale, shift,
                  pw1, pb1, pw2, pb2,
                  ssp_out):
    ot = _onehot_t(seg_ref)
    hn = _apply_bn(h_ref, xphi_ref, ot, rhi, rlo, scale, shift)
    xp = _mlp_rows(hn, pw1[...], pb1[...], pw2[...], pb2[...])
    ssp_out[0] = _otdot(ot, xp)


def _tiny_kernel(ssh_p, ssp_p, sq_p, cnt_p,
                 rw1, rb1, rw2, rb2, bng, bnb,
                 rhi_out, rlo_out, scale_out, shift_out):
    s = jnp.sum(ssh_p[...], axis=0)
    r = _mlp_rows_hi(s, rw1[...], rb1[...], rw2[...], rb2[...])
    c = jnp.sum(cnt_p[...], axis=0)[:, :1]
    g = jnp.sum(ssp_p[...], axis=0)
    s1 = jnp.sum(g + c * r, axis=0, keepdims=True)
    s2 = (jnp.sum(sq_p[...], axis=0) +
          jnp.sum((2.0 * g + c * r) * r, axis=0, keepdims=True))
    mean = s1 / _N
    var = s2 / _N - mean * mean
    sc = bng[...] / jnp.sqrt(var + _EPS)
    rhi, rlo = _split(r)
    rhi_out[...] = rhi
    rlo_out[...] = rlo
    scale_out[...] = sc
    shift_out[...] = bnb[...] - mean * sc


def _out_kernel(ssp_p, qw1, qb1, qw2, qb2, out_ref):
    s = jnp.sum(ssp_p[...], axis=0)
    out_ref[...] = _mlp_rows_hi(s, qw1[...], qb1[...], qw2[...], qb2[...])


def _row_spec():
    return pl.BlockSpec((_B, _D), lambda i: (i, 0))


def _seg_spec():
    return pl.BlockSpec((1, 1, _B), lambda i: (i, 0, 0))


def _const_spec(shape):
    return pl.BlockSpec(shape, lambda i: tuple(0 for _ in shape))


def _mlp_args(p):
    return (p["W1"], p["b1"].reshape(1, -1), p["W2"], p["b2"].reshape(1, -1))


def _mlp_specs():
    return [_const_spec((_D, _D)), _const_spec((1, _D)),
            _const_spec((_D, _D)), _const_spec((1, _D))]


_CP = pltpu.CompilerParams(dimension_semantics=("arbitrary",))


def _first_pass(x, seg3, phi):
    out_shapes = (
        jax.ShapeDtypeStruct((_N, _D), jnp.bfloat16),
        jax.ShapeDtypeStruct((_NB, _NSEG, _D), _F32),
        jax.ShapeDtypeStruct((_NB, _NSEG, _D), _F32),
        jax.ShapeDtypeStruct((_NB, 1, _D), _F32),
        jax.ShapeDtypeStruct((_NB, _NSEG, 128), _F32),
    )
    blk = lambda shape: pl.BlockSpec((1,) + shape, lambda i: (i, 0, 0))
    out_specs = (
        _row_spec(), blk((_NSEG, _D)), blk((_NSEG, _D)),
        blk((1, _D)), blk((_NSEG, 128)),
    )
    return pl.pallas_call(
        _first_kernel,
        grid=(_NB,),
        in_specs=[_row_spec(), _seg_spec()] + _mlp_specs(),
        out_specs=out_specs,
        out_shape=out_shapes,
        compiler_params=pltpu.CompilerParams(
            dimension_semantics=("parallel",)),
    )(x, seg3, *_mlp_args(phi))


def _stat_specs():
    return [_const_spec((_NB, _NSEG, _D)), _const_spec((_NB, _NSEG, _D)),
            _const_spec((_NB, 1, _D)), _const_spec((_NB, _NSEG, 128))]


def _mid_pass(h, xphi, seg3, ssh, ssp, sq, cnt, rho, bng, bnb, phi_next):
    out_shapes = (
        jax.ShapeDtypeStruct((_N, _D), _F32),
        jax.ShapeDtypeStruct((_N, _D), jnp.bfloat16),
        jax.ShapeDtypeStruct((_NB, _NSEG, _D), _F32),
        jax.ShapeDtypeStruct((_NB, _NSEG, _D), _F32),
        jax.ShapeDtypeStruct((_NB, 1, _D), _F32),
    )
    blk = lambda shape: pl.BlockSpec((1,) + shape, lambda i: (i, 0, 0))
    out_specs = (
        _row_spec(), _row_spec(), blk((_NSEG, _D)),
        blk((_NSEG, _D)), blk((1, _D)),
    )
    scratch = [pltpu.VMEM((_NSEG, _D), jnp.bfloat16),
               pltpu.VMEM((_NSEG, _D), jnp.bfloat16),
               pltpu.VMEM((1, _D), _F32), pltpu.VMEM((1, _D), _F32)]
    return pl.pallas_call(
        _mid_kernel,
        grid=(_NB,),
        in_specs=([_row_spec(), _row_spec(), _seg_spec()] + _stat_specs()
                  + _mlp_specs() + [_const_spec((1, _D)), _const_spec((1, _D))]
                  + _mlp_specs()),
        out_specs=out_specs,
        out_shape=out_shapes,
        scratch_shapes=scratch,
        compiler_params=_CP,
    )(h, xphi, seg3, ssh, ssp, sq, cnt, *_mlp_args(rho),
      bng.reshape(1, -1), bnb.reshape(1, -1), *_mlp_args(phi_next))


def _final_pass(h, xphi, seg3, ssh, ssp, sq, cnt, rho, bng, bnb, pool):
    small = pl.pallas_call(
        _tiny_kernel,
        out_shape=(
            jax.ShapeDtypeStruct((_NSEG, _D), jnp.bfloat16),
            jax.ShapeDtypeStruct((_NSEG, _D), jnp.bfloat16),
            jax.ShapeDtypeStruct((1, _D), _F32),
            jax.ShapeDtypeStruct((1, _D), _F32),
        ),
    )(ssh, ssp, sq, cnt, *_mlp_args(rho),
      bng.reshape(1, -1), bnb.reshape(1, -1))
    blk = lambda shape: pl.BlockSpec((1,) + shape, lambda i: (i, 0, 0))
    small_specs = [_const_spec((_NSEG, _D)), _const_spec((_NSEG, _D)),
                   _const_spec((1, _D)), _const_spec((1, _D))]
    ssp_pool = pl.pallas_call(
        _final_kernel,
        grid=(_NB,),
        in_specs=([_row_spec(), _row_spec(), _seg_spec()] + small_specs
                  + _mlp_specs()),
        out_specs=blk((_NSEG, _D)),
        out_shape=jax.ShapeDtypeStruct((_NB, _NSEG, _D), _F32),
        compiler_params=pltpu.CompilerParams(
            dimension_semantics=("parallel",)),
    )(h, xphi, seg3, *small, *_mlp_args(pool["phi"]))
    qspecs = [_const_spec((_D, _D)), _const_spec((1, _D)),
              _const_spec((_D, _DOUT)), _const_spec((1, _DOUT))]
    return pl.pallas_call(
        _out_kernel,
        out_shape=jax.ShapeDtypeStruct((_NSEG, _DOUT), _F32),
    )(ssp_pool, *_mlp_args(pool["rho"]))


def kernel(x, seg, params):
    seg3 = seg.astype(jnp.int32).reshape(_NB, 1, _B)
    layers = params["layers"]
    xphi, ssh, ssp, sq, cnt = _first_pass(x, seg3, layers[0]["phi"])
    h = x
    for li in range(_NLAYERS - 1):
        lyr = layers[li]
        h, xphi, ssh, ssp, sq = _mid_pass(
            h, xphi, seg3, ssh, ssp, sq, cnt,
            lyr["rho"], lyr["bn_g"], lyr["bn_b"], layers[li + 1]["phi"])
    lyr = layers[_NLAYERS - 1]
    return _final_pass(h, xphi, seg3, ssh, ssp, sq, cnt,
                       lyr["rho"], lyr["bn_g"], lyr["bn_b"], params["pooling"])


# speedup vs baseline: 1.0194x; 1.0194x over previous
import jax
import jax.numpy as jnp
from jax.experimental import pallas as pl
from jax.experimental.pallas import tpu as pltpu

_N = 32768
_D = 256
_DOUT = 128
_NSEG = 16
_NLAYERS = 3
_B = 4096
_NB = _N // _B
_EPS = 1e-5
_F32 = jnp.float32


def _mlp_rows(x, w1, b1, w2, b2):
    h = jnp.maximum(_bdot(x, w1) + b1, 0.0)
    return _bdot(h, w2) + b2


def _mlp_rows_hi(x, w1, b1, w2, b2):
    h = jnp.maximum(_bdot(x, w1) + b1, 0.0)
    return _bdot(h, w2) + b2


def _bdot(a, b):
    return jnp.dot(a.astype(jnp.bfloat16), b.astype(jnp.bfloat16),
                   preferred_element_type=_F32)


def _onehot_t(seg_ref):
    sv = seg_ref[0]
    ids = jax.lax.broadcasted_iota(jnp.int32, (_NSEG, _B), 0)
    ot = jnp.where(ids == sv, 1.0, 0.0).astype(_F32)
    return ot.astype(jnp.bfloat16)


def _split(v):
    hi = v.astype(jnp.bfloat16)
    lo = (v - hi.astype(_F32)).astype(jnp.bfloat16)
    return hi, lo


def _otdot(ot, v):
    hi, lo = _split(v)
    return (jnp.dot(ot, lo, preferred_element_type=_F32) +
            jnp.dot(ot, hi, preferred_element_type=_F32))


def _otdot_hi(ot, v):
    return jnp.dot(ot, v.astype(jnp.bfloat16), preferred_element_type=_F32)


def _accum(i, ref, val):
    @pl.when(i == 0)
    def _():
        ref[...] = val

    @pl.when(i > 0)
    def _():
        ref[...] += val


def _stats_step0(i, ssh_in, ssp_in, sq_in, cnt_in, rw1, rb1, rw2, rb2,
                 bng, bnb, rhi_s, rlo_s, scale_s, shift_s):
    @pl.when(i == 0)
    def _():
        s = jnp.sum(ssh_in[...], axis=0)
        r = _mlp_rows_hi(s, rw1[...], rb1[...], rw2[...], rb2[...])
        c = jnp.sum(cnt_in[...], axis=0)[:, :1]
        g = jnp.sum(ssp_in[...], axis=0)
        s1 = jnp.sum(g + c * r, axis=0, keepdims=True)
        s2 = (jnp.sum(sq_in[...], axis=0) +
              jnp.sum((2.0 * g + c * r) * r, axis=0, keepdims=True))
        mean = s1 / _N
        var = s2 / _N - mean * mean
        sc = bng[...] / jnp.sqrt(var + _EPS)
        rhi, rlo = _split(r)
        rhi_s[...] = rhi
        rlo_s[...] = rlo
        scale_s[...] = sc
        shift_s[...] = bnb[...] - mean * sc


def _apply_bn(h_ref, xphi_ref, ot, rhi_s, rlo_s, scale_s, shift_s):
    dn = (((0,), (0,)), ((), ()))
    rr = (jax.lax.dot_general(ot, rlo_s[...], dn, preferred_element_type=_F32) +
          jax.lax.dot_general(ot, rhi_s[...], dn, preferred_element_type=_F32))
    t = xphi_ref[...].astype(_F32) + rr
    return h_ref[...] + t * scale_s[...] + shift_s[...]


def _first_kernel(x_ref, seg_ref, w1, b1, w2, b2,
                  xphi_out, ssh_out, ssp_out, sq_out, cnt_out):
    ot = _onehot_t(seg_ref)
    x = x_ref[...]
    xp = _mlp_rows(x, w1[...], b1[...], w2[...], b2[...])
    xphi_out[...] = xp.astype(jnp.bfloat16)
    ssh_out[0] = _otdot(ot, x)
    ssp_out[0] = _otdot_hi(ot, xp)
    sq_out[0] = jnp.sum(xp * xp, axis=0, keepdims=True)
    cnt_out[0] = jnp.broadcast_to(
        jnp.sum(ot.astype(_F32), axis=1, keepdims=True), (_NSEG, 128))


def _mid_kernel(h_ref, xphi_ref, seg_ref,
                ssh_in, ssp_in, sq_in, cnt_in,
                rw1, rb1, rw2, rb2, bng, bnb,
                pw1, pb1, pw2, pb2,
                h_out, xphi_out, ssh_out, ssp_out, sq_out,
                rhi_s, rlo_s, scale_s, shift_s):
    i = pl.program_id(0)
    _stats_step0(i, ssh_in, ssp_in, sq_in, cnt_in, rw1, rb1, rw2, rb2,
                 bng, bnb, rhi_s, rlo_s, scale_s, shift_s)
    ot = _onehot_t(seg_ref)
    hn = _apply_bn(h_ref, xphi_ref, ot, rhi_s, rlo_s, scale_s, shift_s)
    h_out[...] = hn
    xp = _mlp_rows(hn, pw1[...], pb1[...], pw2[...], pb2[...])
    xphi_out[...] = xp.astype(jnp.bfloat16)
    ssh_out[0] = _otdot(ot, hn)
    ssp_out[0] = _otdot_hi(ot, xp)
    sq_out[0] = jnp.sum(xp * xp, axis=0, keepdims=True)


def _final_kernel(h_ref, xphi_ref, seg_ref,
                  ssh_in, ssp_in, sq_in, cnt_in,
                  rw1, rb1, rw2, rb2, bng, bnb,
                  pw1, pb1, pw2, pb2,
                  qw1, qb1, qw2, qb2,
                  out_ref,
                  rhi_s, rlo_s, scale_s, shift_s, acc_s):
    i = pl.program_id(0)
    _stats_step0(i, ssh_in, ssp_in, sq_in, cnt_in, rw1, rb1, rw2, rb2,
                 bng, bnb, rhi_s, rlo_s, scale_s, shift_s)
    ot = _onehot_t(seg_ref)
    hn = _apply_bn(h_ref, xphi_ref, ot, rhi_s, rlo_s, scale_s, shift_s)
    xp = _mlp_rows(hn, pw1[...], pb1[...], pw2[...], pb2[...])
    _accum(i, acc_s, _otdot(ot, xp))

    @pl.when(i == _NB - 1)
    def _():
        out_ref[...] = _mlp_rows_hi(acc_s[...], qw1[...], qb1[...],
                                 qw2[...], qb2[...])


def _row_spec():
    return pl.BlockSpec((_B, _D), lambda i: (i, 0))


def _seg_spec():
    return pl.BlockSpec((1, 1, _B), lambda i: (i, 0, 0))


def _const_spec(shape):
    return pl.BlockSpec(shape, lambda i: tuple(0 for _ in shape))


def _mlp_args(p):
    return (p["W1"], p["b1"].reshape(1, -1), p["W2"], p["b2"].reshape(1, -1))


def _mlp_specs():
    return [_const_spec((_D, _D)), _const_spec((1, _D)),
            _const_spec((_D, _D)), _const_spec((1, _D))]


_CP = pltpu.CompilerParams(dimension_semantics=("arbitrary",))


def _first_pass(x, seg3, phi):
    out_shapes = (
        jax.ShapeDtypeStruct((_N, _D), jnp.bfloat16),
        jax.ShapeDtypeStruct((_NB, _NSEG, _D), _F32),
        jax.ShapeDtypeStruct((_NB, _NSEG, _D), _F32),
        jax.ShapeDtypeStruct((_NB, 1, _D), _F32),
        jax.ShapeDtypeStruct((_NB, _NSEG, 128), _F32),
    )
    blk = lambda shape: pl.BlockSpec((1,) + shape, lambda i: (i, 0, 0))
    out_specs = (
        _row_spec(), blk((_NSEG, _D)), blk((_NSEG, _D)),
        blk((1, _D)), blk((_NSEG, 128)),
    )
    return pl.pallas_call(
        _first_kernel,
        grid=(_NB,),
        in_specs=[_row_spec(), _seg_spec()] + _mlp_specs(),
        out_specs=out_specs,
        out_shape=out_shapes,
        compiler_params=pltpu.CompilerParams(
            dimension_semantics=("parallel",)),
    )(x, seg3, *_mlp_args(phi))


def _stat_specs():
    return [_const_spec((_NB, _NSEG, _D)), _const_spec((_NB, _NSEG, _D)),
            _const_spec((_NB, 1, _D)), _const_spec((_NB, _NSEG, 128))]


def _mid_pass(h, xphi, seg3, ssh, ssp, sq, cnt, rho, bng, bnb, phi_next):
    out_shapes = (
        jax.ShapeDtypeStruct((_N, _D), _F32),
        jax.ShapeDtypeStruct((_N, _D), jnp.bfloat16),
        jax.ShapeDtypeStruct((_NB, _NSEG, _D), _F32),
        jax.ShapeDtypeStruct((_NB, _NSEG, _D), _F32),
        jax.ShapeDtypeStruct((_NB, 1, _D), _F32),
    )
    blk = lambda shape: pl.BlockSpec((1,) + shape, lambda i: (i, 0, 0))
    out_specs = (
        _row_spec(), _row_spec(), blk((_NSEG, _D)),
        blk((_NSEG, _D)), blk((1, _D)),
    )
    scratch = [pltpu.VMEM((_NSEG, _D), jnp.bfloat16),
               pltpu.VMEM((_NSEG, _D), jnp.bfloat16),
               pltpu.VMEM((1, _D), _F32), pltpu.VMEM((1, _D), _F32)]
    return pl.pallas_call(
        _mid_kernel,
        grid=(_NB,),
        in_specs=([_row_spec(), _row_spec(), _seg_spec()] + _stat_specs()
                  + _mlp_specs() + [_const_spec((1, _D)), _const_spec((1, _D))]
                  + _mlp_specs()),
        out_specs=out_specs,
        out_shape=out_shapes,
        scratch_shapes=scratch,
        compiler_params=_CP,
    )(h, xphi, seg3, ssh, ssp, sq, cnt, *_mlp_args(rho),
      bng.reshape(1, -1), bnb.reshape(1, -1), *_mlp_args(phi_next))


def _final_pass(h, xphi, seg3, ssh, ssp, sq, cnt, rho, bng, bnb, pool):
    scratch = [pltpu.VMEM((_NSEG, _D), jnp.bfloat16),
               pltpu.VMEM((_NSEG, _D), jnp.bfloat16),
               pltpu.VMEM((1, _D), _F32), pltpu.VMEM((1, _D), _F32),
               pltpu.VMEM((_NSEG, _D), _F32)]
    qspecs = [_const_spec((_D, _D)), _const_spec((1, _D)),
              _const_spec((_D, _DOUT)), _const_spec((1, _DOUT))]
    return pl.pallas_call(
        _final_kernel,
        grid=(_NB,),
        in_specs=([_row_spec(), _row_spec(), _seg_spec()] + _stat_specs()
                  + _mlp_specs() + [_const_spec((1, _D)), _const_spec((1, _D))]
                  + _mlp_specs() + qspecs),
        out_specs=_const_spec((_NSEG, _DOUT)),
        out_shape=jax.ShapeDtypeStruct((_NSEG, _DOUT), _F32),
        scratch_shapes=scratch,
        compiler_params=_CP,
    )(h, xphi, seg3, ssh, ssp, sq, cnt, *_mlp_args(rho),
      bng.reshape(1, -1), bnb.reshape(1, -1),
      *_mlp_args(pool["phi"]), *_mlp_args(pool["rho"]))


def kernel(x, seg, params):
    seg3 = seg.astype(jnp.int32).reshape(_NB, 1, _B)
    layers = params["layers"]
    xphi, ssh, ssp, sq, cnt = _first_pass(x, seg3, layers[0]["phi"])
    h = x
    for li in range(_NLAYERS - 1):
        lyr = layers[li]
        h, xphi, ssh, ssp, sq = _mid_pass(
            h, xphi, seg3, ssh, ssp, sq, cnt,
            lyr["rho"], lyr["bn_g"], lyr["bn_b"], layers[li + 1]["phi"])
    lyr = layers[_NLAYERS - 1]
    return _final_pass(h, xphi, seg3, ssh, ssp, sq, cnt,
                       lyr["rho"], lyr["bn_g"], lyr["bn_b"], params["pooling"])


# speedup vs baseline: 1.0206x; 1.0011x over previous
import jax
import jax.numpy as jnp
from jax.experimental import pallas as pl
from jax.experimental.pallas import tpu as pltpu

_N = 32768
_D = 256
_DOUT = 128
_NSEG = 16
_NLAYERS = 3
_B = 4096
_NB = _N // _B
_BF = 8192
_NBF = _N // _BF
_EPS = 1e-5
_F32 = jnp.float32


def _mlp_rows(x, w1, b1, w2, b2):
    h = jnp.maximum(_bdot(x, w1) + b1, 0.0)
    return _bdot(h, w2) + b2


def _bdot(a, b):
    return jnp.dot(a.astype(jnp.bfloat16), b.astype(jnp.bfloat16),
                   preferred_element_type=_F32)


def _onehot_t(seg_ref):
    sv = seg_ref[0]
    b = sv.shape[1]
    ids = jax.lax.broadcasted_iota(jnp.int32, (_NSEG, b), 0)
    ot = jnp.where(ids == sv, 1.0, 0.0).astype(_F32)
    return ot.astype(jnp.bfloat16)


def _split(v):
    hi = v.astype(jnp.bfloat16)
    lo = (v - hi.astype(_F32)).astype(jnp.bfloat16)
    return hi, lo


def _otdot(ot, v):
    hi, lo = _split(v)
    return (jnp.dot(ot, lo, preferred_element_type=_F32) +
            jnp.dot(ot, hi, preferred_element_type=_F32))


def _otdot_hi(ot, v):
    return jnp.dot(ot, v.astype(jnp.bfloat16), preferred_element_type=_F32)


def _accum(i, ref, val):
    @pl.when(i == 0)
    def _():
        ref[...] = val

    @pl.when(i > 0)
    def _():
        ref[...] += val


def _stats_step0(i, ssh_in, ssp_in, sq_in, cnt_in, rw1, rb1, rw2, rb2,
                 bng, bnb, rhi_s, rlo_s, scale_s, shift_s):
    @pl.when(i == 0)
    def _():
        s = jnp.sum(ssh_in[...], axis=0)
        r = _mlp_rows(s, rw1[...], rb1[...], rw2[...], rb2[...])
        c = jnp.sum(cnt_in[...], axis=0)[:, :1]
        g = jnp.sum(ssp_in[...], axis=0)
        s1 = jnp.sum(g + c * r, axis=0, keepdims=True)
        s2 = (jnp.sum(sq_in[...], axis=0) +
              jnp.sum((2.0 * g + c * r) * r, axis=0, keepdims=True))
        mean = s1 / _N
        var = s2 / _N - mean * mean
        sc = bng[...] / jnp.sqrt(var + _EPS)
        rhi, rlo = _split(r)
        rhi_s[...] = rhi
        rlo_s[...] = rlo
        scale_s[...] = sc
        shift_s[...] = bnb[...] - mean * sc


def _apply_bn(h_ref, xphi_ref, ot, rhi_s, rlo_s, scale_s, shift_s):
    dn = (((0,), (0,)), ((), ()))
    rr = (jax.lax.dot_general(ot, rlo_s[...], dn, preferred_element_type=_F32) +
          jax.lax.dot_general(ot, rhi_s[...], dn, preferred_element_type=_F32))
    t = xphi_ref[...].astype(_F32) + rr
    return h_ref[...] + t * scale_s[...] + shift_s[...]


def _first_kernel(x_ref, seg_ref, w1, b1, w2, b2,
                  xphi_out, ssh_out, ssp_out, sq_out, cnt_out):
    ot = _onehot_t(seg_ref)
    x = x_ref[...]
    xp = _mlp_rows(x, w1[...], b1[...], w2[...], b2[...])
    xphi_out[...] = xp.astype(jnp.bfloat16)
    ssh_out[0] = _otdot(ot, x)
    ssp_out[0] = _otdot_hi(ot, xp)
    sq_out[0] = jnp.sum(xp * xp, axis=0, keepdims=True)
    cnt_out[0] = jnp.broadcast_to(
        jnp.sum(ot.astype(_F32), axis=1, keepdims=True), (_NSEG, 128))


def _mid_kernel(h_ref, xphi_ref, seg_ref,
                ssh_in, ssp_in, sq_in, cnt_in,
                rw1, rb1, rw2, rb2, bng, bnb,
                pw1, pb1, pw2, pb2,
                h_out, xphi_out, ssh_out, ssp_out, sq_out,
                rhi_s, rlo_s, scale_s, shift_s):
    i = pl.program_id(0)
    _stats_step0(i, ssh_in, ssp_in, sq_in, cnt_in, rw1, rb1, rw2, rb2,
                 bng, bnb, rhi_s, rlo_s, scale_s, shift_s)
    ot = _onehot_t(seg_ref)
    hn = _apply_bn(h_ref, xphi_ref, ot, rhi_s, rlo_s, scale_s, shift_s)
    h_out[...] = hn
    xp = _mlp_rows(hn, pw1[...], pb1[...], pw2[...], pb2[...])
    xphi_out[...] = xp.astype(jnp.bfloat16)
    ssh_out[0] = _otdot(ot, hn)
    ssp_out[0] = _otdot_hi(ot, xp)
    sq_out[0] = jnp.sum(xp * xp, axis=0, keepdims=True)


def _final_kernel(h_ref, xphi_ref, seg_ref,
                  ssh_in, ssp_in, sq_in, cnt_in,
                  rw1, rb1, rw2, rb2, bng, bnb,
                  pw1, pb1, pw2, pb2,
                  qw1, qb1, qw2, qb2,
                  out_ref,
                  rhi_s, rlo_s, scale_s, shift_s, acc_s):
    i = pl.program_id(0)
    _stats_step0(i, ssh_in, ssp_in, sq_in, cnt_in, rw1, rb1, rw2, rb2,
                 bng, bnb, rhi_s, rlo_s, scale_s, shift_s)
    ot = _onehot_t(seg_ref)
    hn = _apply_bn(h_ref, xphi_ref, ot, rhi_s, rlo_s, scale_s, shift_s)
    xp = _mlp_rows(hn, pw1[...], pb1[...], pw2[...], pb2[...])
    _accum(i, acc_s, _otdot(ot, xp))

    @pl.when(i == _NBF - 1)
    def _():
        out_ref[...] = _mlp_rows(acc_s[...], qw1[...], qb1[...],
                                 qw2[...], qb2[...])


def _row_spec(b=_B):
    return pl.BlockSpec((b, _D), lambda i: (i, 0))


def _seg_spec(b=_B):
    return pl.BlockSpec((1, 1, b), lambda i: (i, 0, 0))


def _const_spec(shape):
    return pl.BlockSpec(shape, lambda i: tuple(0 for _ in shape))


def _mlp_args(p):
    return (p["W1"], p["b1"].reshape(1, -1), p["W2"], p["b2"].reshape(1, -1))


def _mlp_specs():
    return [_const_spec((_D, _D)), _const_spec((1, _D)),
            _const_spec((_D, _D)), _const_spec((1, _D))]


_CP = pltpu.CompilerParams(dimension_semantics=("arbitrary",))


def _first_pass(x, seg3, phi):
    out_shapes = (
        jax.ShapeDtypeStruct((_N, _D), jnp.bfloat16),
        jax.ShapeDtypeStruct((_NBF, _NSEG, _D), _F32),
        jax.ShapeDtypeStruct((_NBF, _NSEG, _D), _F32),
        jax.ShapeDtypeStruct((_NBF, 1, _D), _F32),
        jax.ShapeDtypeStruct((_NBF, _NSEG, 128), _F32),
    )
    blk = lambda shape: pl.BlockSpec((1,) + shape, lambda i: (i, 0, 0))
    out_specs = (
        _row_spec(_BF), blk((_NSEG, _D)), blk((_NSEG, _D)),
        blk((1, _D)), blk((_NSEG, 128)),
    )
    return pl.pallas_call(
        _first_kernel,
        grid=(_NBF,),
        in_specs=[_row_spec(_BF), _seg_spec(_BF)] + _mlp_specs(),
        out_specs=out_specs,
        out_shape=out_shapes,
        compiler_params=pltpu.CompilerParams(
            dimension_semantics=("parallel",)),
    )(x, seg3, *_mlp_args(phi))


def _stat_specs(nb, nbc):
    return [_const_spec((nb, _NSEG, _D)), _const_spec((nb, _NSEG, _D)),
            _const_spec((nb, 1, _D)), _const_spec((nbc, _NSEG, 128))]


def _mid_pass(h, xphi, seg3, ssh, ssp, sq, cnt, rho, bng, bnb, phi_next):
    out_shapes = (
        jax.ShapeDtypeStruct((_N, _D), _F32),
        jax.ShapeDtypeStruct((_N, _D), jnp.bfloat16),
        jax.ShapeDtypeStruct((_NB, _NSEG, _D), _F32),
        jax.ShapeDtypeStruct((_NB, _NSEG, _D), _F32),
        jax.ShapeDtypeStruct((_NB, 1, _D), _F32),
    )
    blk = lambda shape: pl.BlockSpec((1,) + shape, lambda i: (i, 0, 0))
    out_specs = (
        _row_spec(), _row_spec(), blk((_NSEG, _D)),
        blk((_NSEG, _D)), blk((1, _D)),
    )
    scratch = [pltpu.VMEM((_NSEG, _D), jnp.bfloat16),
               pltpu.VMEM((_NSEG, _D), jnp.bfloat16),
               pltpu.VMEM((1, _D), _F32), pltpu.VMEM((1, _D), _F32)]
    return pl.pallas_call(
        _mid_kernel,
        grid=(_NB,),
        in_specs=([_row_spec(), _row_spec(), _seg_spec()]
                  + _stat_specs(ssh.shape[0], cnt.shape[0])
                  + _mlp_specs() + [_const_spec((1, _D)), _const_spec((1, _D))]
                  + _mlp_specs()),
        out_specs=out_specs,
        out_shape=out_shapes,
        scratch_shapes=scratch,
        compiler_params=_CP,
    )(h, xphi, seg3, ssh, ssp, sq, cnt, *_mlp_args(rho),
      bng.reshape(1, -1), bnb.reshape(1, -1), *_mlp_args(phi_next))


def _final_pass(h, xphi, seg3, ssh, ssp, sq, cnt, rho, bng, bnb, pool):
    scratch = [pltpu.VMEM((_NSEG, _D), jnp.bfloat16),
               pltpu.VMEM((_NSEG, _D), jnp.bfloat16),
               pltpu.VMEM((1, _D), _F32), pltpu.VMEM((1, _D), _F32),
               pltpu.VMEM((_NSEG, _D), _F32)]
    qspecs = [_const_spec((_D, _D)), _const_spec((1, _D)),
              _const_spec((_D, _DOUT)), _const_spec((1, _DOUT))]
    return pl.pallas_call(
        _final_kernel,
        grid=(_NBF,),
        in_specs=([_row_spec(_BF), _row_spec(_BF), _seg_spec(_BF)]
                  + _stat_specs(ssh.shape[0], cnt.shape[0])
                  + _mlp_specs() + [_const_spec((1, _D)), _const_spec((1, _D))]
                  + _mlp_specs() + qspecs),
        out_specs=_const_spec((_NSEG, _DOUT)),
        out_shape=jax.ShapeDtypeStruct((_NSEG, _DOUT), _F32),
        scratch_shapes=scratch,
        compiler_params=_CP,
    )(h, xphi, seg3, ssh, ssp, sq, cnt, *_mlp_args(rho),
      bng.reshape(1, -1), bnb.reshape(1, -1),
      *_mlp_args(pool["phi"]), *_mlp_args(pool["rho"]))


def kernel(x, seg, params):
    seg32 = seg.astype(jnp.int32)
    seg3 = seg32.reshape(_NB, 1, _B)
    seg3f = seg32.reshape(_NBF, 1, _BF)
    layers = params["layers"]
    xphi, ssh, ssp, sq, cnt = _first_pass(x, seg3f, layers[0]["phi"])
    h = x
    for li in range(_NLAYERS - 1):
        lyr = layers[li]
        h, xphi, ssh, ssp, sq = _mid_pass(
            h, xphi, seg3, ssh, ssp, sq, cnt,
            lyr["rho"], lyr["bn_g"], lyr["bn_b"], layers[li + 1]["phi"])
    lyr = layers[_NLAYERS - 1]
    return _final_pass(h, xphi, seg3f, ssh, ssp, sq, cnt,
                       lyr["rho"], lyr["bn_g"], lyr["bn_b"], params["pooling"])
